# Initial kernel scaffold; baseline (speedup 1.0000x reference)
import jax
import jax.numpy as jnp
from jax.experimental import pallas as pl


def kernel(values, coords, new_coords, shift):
    raise NotImplementedError("write your pallas kernel here")


# speedup vs baseline: 13.8768x; 13.8768x over previous
import functools

import jax
import jax.numpy as jnp
from jax import lax
from jax.experimental import pallas as pl
from jax.experimental.pallas import tpu as pltpu
from jax.experimental.pallas import tpu_sc as plsc

N_IN = 2048
N_TOTAL = 8192
C = 128
K = 4


_QB = 256


def _topk_body(qx_ref, qy_ref, kx_ref, ky_ref, idx_ref):
    dx = qx_ref[...] - kx_ref[...]
    dy = qy_ref[...] - ky_ref[...]
    d = jnp.sqrt(dx * dx + dy * dy)
    iota = lax.broadcasted_iota(jnp.int32, (_QB, N_IN), 1)
    cols = []
    for _ in range(K):
        m = jnp.min(d, axis=1, keepdims=True)
        j = jnp.min(jnp.where(d == m, iota, N_IN), axis=1, keepdims=True)
        d = jnp.where(iota == j, jnp.float32(jnp.inf), d)
        cols.append(j)
    idx_ref[...] = jnp.concatenate(cols, axis=1)


def _tc_topk(qx, qy, kx, ky):
    grid = N_TOTAL // _QB
    return pl.pallas_call(
        _topk_body,
        grid=(grid,),
        in_specs=[
            pl.BlockSpec((_QB, 1), lambda i: (i, 0)),
            pl.BlockSpec((_QB, 1), lambda i: (i, 0)),
            pl.BlockSpec((1, N_IN), lambda i: (0, 0)),
            pl.BlockSpec((1, N_IN), lambda i: (0, 0)),
        ],
        out_specs=pl.BlockSpec((_QB, K), lambda i: (i, 0)),
        out_shape=jax.ShapeDtypeStruct((N_TOTAL, K), jnp.int32),
    )(qx, qy, kx, ky)


_NC = 2
_NS = 16
_NW = _NC * _NS
_QPW = N_TOTAL // _NW
_QCHUNK = 32
_NCHUNK = _QPW // _QCHUNK


def _gather_mean_body(table_hbm, idx_hbm, out_hbm, idx_v, rows_v, out_v, sem):
    c = lax.axis_index("c")
    s = lax.axis_index("s")
    wid = s * _NC + c
    base_q = wid * _QPW

    def chunk_body(ch, carry):
        qb = base_q + ch * _QCHUNK
        pltpu.sync_copy(idx_hbm.at[pl.ds(qb * K, _QCHUNK * K)], idx_v)
        pltpu.async_copy(table_hbm.at[idx_v], rows_v, sem).wait()

        def q_body(q, carry2):
            for l in range(C // 16):
                sl = pl.ds(l * 16, 16)
                acc = (rows_v[K * q, sl] + rows_v[K * q + 1, sl]
                       + rows_v[K * q + 2, sl] + rows_v[K * q + 3, sl])
                out_v[q, sl] = acc * jnp.float32(1.0 / K)
            return carry2

        lax.fori_loop(0, _QCHUNK, q_body, 0, unroll=4)
        pltpu.sync_copy(out_v, out_hbm.at[pl.ds(qb, _QCHUNK)])
        return carry

    lax.fori_loop(0, _NCHUNK, chunk_body, 0)


def _sc_gather_mean(table, idx_flat):
    mesh = plsc.VectorSubcoreMesh(core_axis_name="c", subcore_axis_name="s")
    kern = pl.kernel(
        _gather_mean_body,
        out_type=jax.ShapeDtypeStruct((N_TOTAL, C), jnp.float32),
        mesh=mesh,
        scratch_types=[
            pltpu.VMEM((_QCHUNK * K,), jnp.int32),
            pltpu.VMEM((_QCHUNK * K, C), jnp.float32),
            pltpu.VMEM((_QCHUNK, C), jnp.float32),
            pltpu.SemaphoreType.DMA,
        ],
    )
    return kern(table, idx_flat)


def kernel(values, coords, new_coords, shift):
    all_coords = jnp.concatenate([coords, new_coords], axis=0)
    q = all_coords - shift
    qx = q[:, 0:1]
    qy = q[:, 1:2]
    kx = coords[:, 0][None, :]
    ky = coords[:, 1][None, :]
    idx = _tc_topk(qx, qy, kx, ky)
    table = values.T
    out_rows = _sc_gather_mean(table, idx.reshape(-1))
    return out_rows.T


# speedup vs baseline: 16.3925x; 1.1813x over previous
import functools

import jax
import jax.numpy as jnp
from jax import lax
from jax.experimental import pallas as pl
from jax.experimental.pallas import tpu as pltpu
from jax.experimental.pallas import tpu_sc as plsc

N_IN = 2048
N_TOTAL = 8192
C = 128
K = 4


_QB = 256


def _topk_body(qx_ref, qy_ref, kx_ref, ky_ref, idx_ref):
    dx = qx_ref[...] - kx_ref[...]
    dy = qy_ref[...] - ky_ref[...]
    d = jnp.sqrt(dx * dx + dy * dy)
    iota_f = lax.broadcasted_iota(jnp.int32, (_QB, N_IN), 1).astype(jnp.float32)
    cols = []
    for _ in range(K):
        m = jnp.min(d, axis=1, keepdims=True)
        j = jnp.min(jnp.where(d == m, iota_f, jnp.float32(N_IN)),
                    axis=1, keepdims=True)
        d = jnp.where(iota_f == j, jnp.float32(jnp.inf), d)
        cols.append(j)
    idx_ref[...] = jnp.concatenate(cols, axis=1).astype(jnp.int32)


def _tc_topk(qx, qy, kx, ky):
    grid = N_TOTAL // _QB
    return pl.pallas_call(
        _topk_body,
        grid=(grid,),
        in_specs=[
            pl.BlockSpec((_QB, 1), lambda i: (i, 0)),
            pl.BlockSpec((_QB, 1), lambda i: (i, 0)),
            pl.BlockSpec((1, N_IN), lambda i: (0, 0)),
            pl.BlockSpec((1, N_IN), lambda i: (0, 0)),
        ],
        out_specs=pl.BlockSpec((_QB, K), lambda i: (i, 0)),
        out_shape=jax.ShapeDtypeStruct((N_TOTAL, K), jnp.int32),
    )(qx, qy, kx, ky)


_NC = 2
_NS = 16
_NW = _NC * _NS
_QPW = N_TOTAL // _NW
_QCHUNK = 32
_NCHUNK = _QPW // _QCHUNK


def _gather_mean_body(table_hbm, idx_hbm, out_hbm,
                      idx_v0, idx_v1, rows_v0, rows_v1, out_v, sem0, sem1):
    c = lax.axis_index("c")
    s = lax.axis_index("s")
    wid = s * _NC + c
    base_q = wid * _QPW
    idx_bufs = (idx_v0, idx_v1)
    row_bufs = (rows_v0, rows_v1)
    sems = (sem0, sem1)

    def start(ch):
        qb = base_q + ch * _QCHUNK
        p = ch % 2
        pltpu.sync_copy(idx_hbm.at[pl.ds(qb * K, _QCHUNK * K)], idx_bufs[p])
        return pltpu.async_copy(table_hbm.at[idx_bufs[p]], row_bufs[p], sems[p])

    copies = [start(0)]
    for ch in range(_NCHUNK):
        if ch + 1 < _NCHUNK:
            copies.append(start(ch + 1))
        copies[ch].wait()
        rows_v = row_bufs[ch % 2]
        qb = base_q + ch * _QCHUNK

        def q_body(q, carry2):
            for l in range(C // 16):
                sl = pl.ds(l * 16, 16)
                acc = (rows_v[K * q, sl] + rows_v[K * q + 1, sl]
                       + rows_v[K * q + 2, sl] + rows_v[K * q + 3, sl])
                out_v[q, sl] = acc * jnp.float32(1.0 / K)
            return carry2

        lax.fori_loop(0, _QCHUNK, q_body, 0, unroll=4)
        pltpu.sync_copy(out_v, out_hbm.at[pl.ds(qb, _QCHUNK)])


def _sc_gather_mean(table, idx_flat):
    mesh = plsc.VectorSubcoreMesh(core_axis_name="c", subcore_axis_name="s")
    kern = pl.kernel(
        _gather_mean_body,
        out_type=jax.ShapeDtypeStruct((N_TOTAL, C), jnp.float32),
        mesh=mesh,
        scratch_types=[
            pltpu.VMEM((_QCHUNK * K,), jnp.int32),
            pltpu.VMEM((_QCHUNK * K,), jnp.int32),
            pltpu.VMEM((_QCHUNK * K, C), jnp.float32),
            pltpu.VMEM((_QCHUNK * K, C), jnp.float32),
            pltpu.VMEM((_QCHUNK, C), jnp.float32),
            pltpu.SemaphoreType.DMA,
            pltpu.SemaphoreType.DMA,
        ],
    )
    return kern(table, idx_flat)


def kernel(values, coords, new_coords, shift):
    all_coords = jnp.concatenate([coords, new_coords], axis=0)
    q = all_coords - shift
    qx = q[:, 0:1]
    qy = q[:, 1:2]
    kx = coords[:, 0][None, :]
    ky = coords[:, 1][None, :]
    idx = _tc_topk(qx, qy, kx, ky)
    table = values.T
    out_rows = _sc_gather_mean(table, idx.reshape(-1))
    return out_rows.T
